# baseline (speedup 1.0000x reference)
import functools

import jax
import jax.numpy as jnp
from jax import lax
from jax.experimental import pallas as pl
from jax.experimental.pallas import tpu as pltpu
from jax.experimental.pallas import tpu_sc as plsc

_H = 128
_R = 8
_NB = 4
_N = 10000
_E = 320000
_K = _N * _R
_KP = 81920
_NC = 2
_NS = 16
_NW = _NC * _NS
_EPW = _E // _NW
_CH = 80
_NCH = _EPW // _CH
_CH1 = 2000
_NP = 10240
_RPT = _NP // _NS
_L = 16


def _splat(vec, e):
    idx = jnp.full((_L,), e, dtype=jnp.int32)
    return lax.gather(
        vec, idx[:, None],
        dimension_numbers=lax.GatherDimensionNumbers(
            offset_dims=(), collapsed_slice_dims=(0,), start_index_map=(0,)),
        slice_sizes=(1,), mode=lax.GatherScatterMode.PROMISE_IN_BOUNDS)


def _sc_cnt_body(dst_hbm, typ_hbm, zk_hbm, out_hbm, cnt_v, dst_v, typ_v):
    cid = lax.axis_index("c")
    sid = lax.axis_index("s")
    wid = sid * _NC + cid
    pltpu.sync_copy(zk_hbm, cnt_v)

    def chunk(c, _):
        off = wid * _EPW + c * _CH1
        pltpu.sync_copy(dst_hbm.at[pl.ds(off, _CH1)], dst_v)
        pltpu.sync_copy(typ_hbm.at[pl.ds(off, _CH1)], typ_v)

        def group(j, _):
            d = dst_v[pl.ds(j * _L, _L)]
            t = typ_v[pl.ds(j * _L, _L)]
            key = t * _N + d
            plsc.addupdate_scatter(cnt_v, [key],
                                   jnp.ones((_L,), jnp.float32))
            return 0

        return lax.fori_loop(0, _CH1 // _L, group, 0)

    lax.fori_loop(0, _EPW // _CH1, chunk, 0)
    pltpu.sync_copy(cnt_v, out_hbm.at[wid])


def _sc_cnt(dst, typ, zeros_k):
    mesh = plsc.VectorSubcoreMesh(core_axis_name="c", subcore_axis_name="s")
    return pl.kernel(
        _sc_cnt_body,
        out_type=jax.ShapeDtypeStruct((_NW, _KP), jnp.float32),
        mesh=mesh,
        compiler_params=pltpu.CompilerParams(needs_layout_passes=False),
        scratch_types=[
            pltpu.VMEM((_KP,), jnp.float32),
            pltpu.VMEM((_CH1,), jnp.int32),
            pltpu.VMEM((_CH1,), jnp.int32),
        ],
    )(dst, typ, zeros_k)


def _tc_mm_body(x_ref, bases_ref, comp_ref, root_ref, bias_ref, y_ref, o_ref):
    x = x_ref[...]
    zs = [jnp.dot(x, bases_ref[b], preferred_element_type=jnp.float32)
          for b in range(_NB)]
    for r in range(_R):
        acc = zs[0] * comp_ref[r, 0]
        for b in range(1, _NB):
            acc = acc + zs[b] * comp_ref[r, b]
        y_ref[r, :, :] = acc
    o_ref[...] = (jnp.dot(x, root_ref[...], preferred_element_type=jnp.float32)
                  + bias_ref[...])


def _tc_mm(x, bases, comp, root, bias):
    blk = 1000
    return pl.pallas_call(
        _tc_mm_body,
        grid=(_N // blk,),
        in_specs=[
            pl.BlockSpec((blk, _H), lambda i: (i, 0)),
            pl.BlockSpec((_NB, _H, _H), lambda i: (0, 0, 0)),
            pl.BlockSpec(memory_space=pltpu.SMEM),
            pl.BlockSpec((_H, _H), lambda i: (0, 0)),
            pl.BlockSpec((_H,), lambda i: (0,)),
        ],
        out_specs=[
            pl.BlockSpec((_R, blk, _H), lambda i: (0, i, 0)),
            pl.BlockSpec((blk, _H), lambda i: (i, 0)),
        ],
        out_shape=[
            jax.ShapeDtypeStruct((_R, _N, _H), jnp.float32),
            jax.ShapeDtypeStruct((_N, _H), jnp.float32),
        ],
    )(x, bases, comp, root, bias)


def _tc_inv_body(p_ref, inv_ref):
    s = jnp.sum(p_ref[...], axis=0)
    inv_ref[...] = 1.0 / jnp.maximum(s, 1.0)


def _tc_inv(parts):
    blk = 16384
    return pl.pallas_call(
        _tc_inv_body,
        grid=(_KP // blk,),
        in_specs=[pl.BlockSpec((_NW, blk), lambda i: (0, i))],
        out_specs=pl.BlockSpec((blk,), lambda i: (i,)),
        out_shape=jax.ShapeDtypeStruct((_KP,), jnp.float32),
    )(parts)


def _sc_scale_body(dst_hbm, typ_hbm, inv_hbm, out_hbm, inv_v, dst_v, typ_v,
                   scl_v):
    cid = lax.axis_index("c")
    sid = lax.axis_index("s")
    wid = sid * _NC + cid
    pltpu.sync_copy(inv_hbm, inv_v)

    def chunk(c, _):
        off = wid * _EPW + c * _CH1
        pltpu.sync_copy(dst_hbm.at[pl.ds(off, _CH1)], dst_v)
        pltpu.sync_copy(typ_hbm.at[pl.ds(off, _CH1)], typ_v)

        def group(j, _):
            sl = pl.ds(j * _L, _L)
            key = typ_v[sl] * _N + dst_v[sl]
            scl_v[sl] = plsc.load_gather(inv_v, [key])
            return 0

        lax.fori_loop(0, _CH1 // _L, group, 0)
        pltpu.sync_copy(scl_v, out_hbm.at[pl.ds(off, _CH1)])
        return 0

    lax.fori_loop(0, _EPW // _CH1, chunk, 0)


def _sc_scale(dst, typ, inv):
    mesh = plsc.VectorSubcoreMesh(core_axis_name="c", subcore_axis_name="s")
    return pl.kernel(
        _sc_scale_body,
        out_type=jax.ShapeDtypeStruct((_E,), jnp.float32),
        mesh=mesh,
        compiler_params=pltpu.CompilerParams(needs_layout_passes=False),
        scratch_types=[
            pltpu.VMEM((_KP,), jnp.float32),
            pltpu.VMEM((_CH1,), jnp.int32),
            pltpu.VMEM((_CH1,), jnp.int32),
            pltpu.VMEM((_CH1,), jnp.float32),
        ],
    )(dst, typ, inv)


def _sc_edge_body(src_hbm, dst_hbm, typ_hbm, scl_hbm, y_hbm, znh_hbm,
                  out_hbm, acc, rows0, rows1, gkey_v, dst0, dst1, scl0,
                  scl1, st_s, st_t, sem0, sem1, ss0, ss1):
    cid = lax.axis_index("c")
    sid = lax.axis_index("s")
    wid = sid * _NC + cid
    base_e = wid * _EPW

    for k in range(_RPT // 64):
        rows = pl.ds(sid * _RPT + k * 64, 64)
        pltpu.sync_copy(znh_hbm.at[rows], acc.at[rows])

    def sup(c2, _):
        off = base_e + c2 * _CH1
        pltpu.sync_copy(src_hbm.at[pl.ds(off, _CH1)], st_s)
        pltpu.sync_copy(typ_hbm.at[pl.ds(off, _CH1)], st_t)

        def fill(kk, _):
            for j2 in range(5):
                sl = pl.ds(kk * _CH + j2 * _L, _L)
                gkey_v[pl.ds(c2 * _CH1 + kk * _CH + j2 * _L, _L)] = (
                    st_t[sl] * _N + st_s[sl])
            return 0

        return lax.fori_loop(0, _CH1 // _CH, fill, 0)

    lax.fori_loop(0, _EPW // _CH1, sup, 0)
    plsc.subcore_barrier()

    def start_chunk(k, rows_ref, dst_ref, scl_ref, sem):
        off = base_e + k * _CH
        pltpu.async_copy(dst_hbm.at[pl.ds(off, _CH)], dst_ref, sem)
        pltpu.async_copy(scl_hbm.at[pl.ds(off, _CH)], scl_ref, sem)
        pltpu.async_copy(y_hbm.at[gkey_v.at[pl.ds(k * _CH, _CH)]],
                         rows_ref, sem)

    def wait_chunk(k, rows_ref, dst_ref, scl_ref, sem):
        off = base_e + k * _CH
        pltpu.make_async_copy(dst_hbm.at[pl.ds(off, _CH)], dst_ref,
                              sem).wait()
        pltpu.make_async_copy(scl_hbm.at[pl.ds(off, _CH)], scl_ref,
                              sem).wait()
        pltpu.make_async_copy(y_hbm.at[gkey_v.at[pl.ds(k * _CH, _CH)]],
                              rows_ref, sem).wait()

    def scale_rows(rows_ref, scl_ref):
        for j in range(_CH // _L):
            iv = scl_ref[pl.ds(j * _L, _L)]
            for e in range(_L):
                row = j * _L + e
                sc = _splat(iv, e)
                for q in range(_H // _L):
                    rows_ref[row, pl.ds(q * _L, _L)] = (
                        rows_ref[row, pl.ds(q * _L, _L)] * sc)

    def start_scatter(rows_ref, dst_ref, sem):
        pltpu.async_copy(rows_ref, acc.at[dst_ref], sem, add=True)

    def wait_scatter(rows_ref, dst_ref, sem):
        pltpu.make_async_copy(rows_ref, acc.at[dst_ref], sem).wait()

    start_chunk(0, rows0, dst0, scl0, sem0)
    start_chunk(1, rows1, dst1, scl1, sem1)

    def body(i, _):
        k0 = 2 * i
        k1 = 2 * i + 1
        wait_chunk(k0, rows0, dst0, scl0, sem0)
        scale_rows(rows0, scl0)
        start_scatter(rows0, dst0, ss0)
        wait_chunk(k1, rows1, dst1, scl1, sem1)
        scale_rows(rows1, scl1)
        start_scatter(rows1, dst1, ss1)
        wait_scatter(rows0, dst0, ss0)

        @pl.when(k0 + 2 < _NCH)
        def _():
            start_chunk(k0 + 2, rows0, dst0, scl0, sem0)

        wait_scatter(rows1, dst1, ss1)

        @pl.when(k1 + 2 < _NCH)
        def _():
            start_chunk(k1 + 2, rows1, dst1, scl1, sem1)

        return 0

    lax.fori_loop(0, _NCH // 2, body, 0)
    wait_chunk(_NCH - 1, rows0, dst0, scl0, sem0)
    scale_rows(rows0, scl0)
    pltpu.sync_copy(rows0, acc.at[dst0], add=True)

    plsc.subcore_barrier()
    for k in range(_RPT // 64):
        rows = pl.ds(sid * _RPT + k * 64, 64)
        pltpu.sync_copy(acc.at[rows], out_hbm.at[cid, rows])


def _sc_edges(src, dst, typ, scl, y2, zeros_nh):
    mesh = plsc.VectorSubcoreMesh(core_axis_name="c", subcore_axis_name="s")
    return pl.kernel(
        _sc_edge_body,
        out_type=jax.ShapeDtypeStruct((_NC, _NP, _H), jnp.float32),
        mesh=mesh,
        compiler_params=pltpu.CompilerParams(needs_layout_passes=False),
        scratch_types=[
            pltpu.VMEM_SHARED((_NP, _H), jnp.float32),
            pltpu.VMEM((_CH, _H), jnp.float32),
            pltpu.VMEM((_CH, _H), jnp.float32),
            pltpu.VMEM((_EPW,), jnp.int32),
            pltpu.VMEM((_CH,), jnp.int32),
            pltpu.VMEM((_CH,), jnp.int32),
            pltpu.VMEM((_CH,), jnp.float32),
            pltpu.VMEM((_CH,), jnp.float32),
            pltpu.VMEM((_CH1,), jnp.int32),
            pltpu.VMEM((_CH1,), jnp.int32),
            pltpu.SemaphoreType.DMA,
            pltpu.SemaphoreType.DMA,
            pltpu.SemaphoreType.DMA,
            pltpu.SemaphoreType.DMA,
        ],
    )(src, dst, typ, scl, y2, zeros_nh)


def _tc_combine_body(o_ref, p_ref, out_ref):
    out_ref[...] = o_ref[...] + p_ref[0] + p_ref[1]


def _tc_combine(out0, parts):
    blk = 1000
    return pl.pallas_call(
        _tc_combine_body,
        grid=(_N // blk,),
        in_specs=[
            pl.BlockSpec((blk, _H), lambda i: (i, 0)),
            pl.BlockSpec((_NC, blk, _H), lambda i: (0, i, 0)),
        ],
        out_specs=pl.BlockSpec((blk, _H), lambda i: (i, 0)),
        out_shape=jax.ShapeDtypeStruct((_N, _H), jnp.float32),
    )(out0, parts)


def kernel(input_s, input_a, edge_index, edge_type, bases, comp, root, bias):
    x = jnp.concatenate([input_s, input_a], axis=1).reshape(-1, _H)
    src = edge_index[0]
    dst = edge_index[1]
    typ = edge_type

    zeros_k = jnp.zeros((_KP,), jnp.float32)
    zeros_nh = jnp.zeros((_NP, _H), jnp.float32)

    cnt_parts = _sc_cnt(dst, typ, zeros_k)
    y, out0 = _tc_mm(x, bases, comp, root, bias)
    inv = _tc_inv(cnt_parts)
    scl = _sc_scale(dst, typ, inv)
    parts = _sc_edges(src, dst, typ, scl, y.reshape(_K, _H), zeros_nh)
    out = _tc_combine(out0, parts)

    dtrp = out.reshape(input_s.shape[0], -1, _H)
    sent, act = jnp.split(dtrp, 2, axis=1)
    return (sent, act)


# speedup vs baseline: 1.0105x; 1.0105x over previous
import functools

import jax
import jax.numpy as jnp
from jax import lax
from jax.experimental import pallas as pl
from jax.experimental.pallas import tpu as pltpu
from jax.experimental.pallas import tpu_sc as plsc

_H = 128
_R = 8
_NB = 4
_N = 10000
_E = 320000
_K = _N * _R
_KP = 81920
_NC = 2
_NS = 16
_NW = _NC * _NS
_EPW = _E // _NW
_CH = 80
_NCH = _EPW // _CH
_CH1 = 2000
_NP = 10240
_RPT = _NP // _NS
_L = 16


def _splat(vec, e):
    idx = jnp.full((_L,), e, dtype=jnp.int32)
    return lax.gather(
        vec, idx[:, None],
        dimension_numbers=lax.GatherDimensionNumbers(
            offset_dims=(), collapsed_slice_dims=(0,), start_index_map=(0,)),
        slice_sizes=(1,), mode=lax.GatherScatterMode.PROMISE_IN_BOUNDS)


def _sc_cnt_body(dst_hbm, typ_hbm, zk_hbm, out_hbm, cnt_v, dst_v, typ_v):
    cid = lax.axis_index("c")
    sid = lax.axis_index("s")
    wid = sid * _NC + cid
    pltpu.sync_copy(zk_hbm, cnt_v)

    def chunk(c, _):
        off = wid * _EPW + c * _CH1
        pltpu.sync_copy(dst_hbm.at[pl.ds(off, _CH1)], dst_v)
        pltpu.sync_copy(typ_hbm.at[pl.ds(off, _CH1)], typ_v)

        def group(j, _):
            d = dst_v[pl.ds(j * _L, _L)]
            t = typ_v[pl.ds(j * _L, _L)]
            key = t * _N + d
            plsc.addupdate_scatter(cnt_v, [key],
                                   jnp.ones((_L,), jnp.float32))
            return 0

        return lax.fori_loop(0, _CH1 // _L, group, 0)

    lax.fori_loop(0, _EPW // _CH1, chunk, 0)
    pltpu.sync_copy(cnt_v, out_hbm.at[wid])


def _sc_cnt(dst, typ, zeros_k):
    mesh = plsc.VectorSubcoreMesh(core_axis_name="c", subcore_axis_name="s")
    return pl.kernel(
        _sc_cnt_body,
        out_type=jax.ShapeDtypeStruct((_NW, _KP), jnp.float32),
        mesh=mesh,
        compiler_params=pltpu.CompilerParams(needs_layout_passes=False),
        scratch_types=[
            pltpu.VMEM((_KP,), jnp.float32),
            pltpu.VMEM((_CH1,), jnp.int32),
            pltpu.VMEM((_CH1,), jnp.int32),
        ],
    )(dst, typ, zeros_k)


def _tc_mm_body(x_ref, bases_ref, comp_ref, root_ref, bias_ref, y_ref, o_ref):
    x = x_ref[...]
    zs = [jnp.dot(x, bases_ref[b], preferred_element_type=jnp.float32)
          for b in range(_NB)]
    for r in range(_R):
        acc = zs[0] * comp_ref[r, 0]
        for b in range(1, _NB):
            acc = acc + zs[b] * comp_ref[r, b]
        y_ref[r, :, :] = acc
    o_ref[...] = (jnp.dot(x, root_ref[...], preferred_element_type=jnp.float32)
                  + bias_ref[...])


def _tc_mm(x, bases, comp, root, bias):
    blk = 1000
    return pl.pallas_call(
        _tc_mm_body,
        grid=(_N // blk,),
        in_specs=[
            pl.BlockSpec((blk, _H), lambda i: (i, 0)),
            pl.BlockSpec((_NB, _H, _H), lambda i: (0, 0, 0)),
            pl.BlockSpec(memory_space=pltpu.SMEM),
            pl.BlockSpec((_H, _H), lambda i: (0, 0)),
            pl.BlockSpec((_H,), lambda i: (0,)),
        ],
        out_specs=[
            pl.BlockSpec((_R, blk, _H), lambda i: (0, i, 0)),
            pl.BlockSpec((blk, _H), lambda i: (i, 0)),
        ],
        out_shape=[
            jax.ShapeDtypeStruct((_R, _N, _H), jnp.float32),
            jax.ShapeDtypeStruct((_N, _H), jnp.float32),
        ],
    )(x, bases, comp, root, bias)


def _tc_inv_body(p_ref, inv_ref):
    s = jnp.sum(p_ref[...], axis=0)
    inv_ref[...] = 1.0 / jnp.maximum(s, 1.0)


def _tc_inv(parts):
    blk = 16384
    return pl.pallas_call(
        _tc_inv_body,
        grid=(_KP // blk,),
        in_specs=[pl.BlockSpec((_NW, blk), lambda i: (0, i))],
        out_specs=pl.BlockSpec((blk,), lambda i: (i,)),
        out_shape=jax.ShapeDtypeStruct((_KP,), jnp.float32),
    )(parts)


def _sc_scale_body(dst_hbm, typ_hbm, inv_hbm, out_hbm, inv_v, dst_v, typ_v,
                   scl_v):
    cid = lax.axis_index("c")
    sid = lax.axis_index("s")
    wid = sid * _NC + cid
    pltpu.sync_copy(inv_hbm, inv_v)

    def chunk(c, _):
        off = wid * _EPW + c * _CH1
        pltpu.sync_copy(dst_hbm.at[pl.ds(off, _CH1)], dst_v)
        pltpu.sync_copy(typ_hbm.at[pl.ds(off, _CH1)], typ_v)

        def group(j, _):
            sl = pl.ds(j * _L, _L)
            key = typ_v[sl] * _N + dst_v[sl]
            scl_v[sl] = plsc.load_gather(inv_v, [key])
            return 0

        lax.fori_loop(0, _CH1 // _L, group, 0)
        pltpu.sync_copy(scl_v, out_hbm.at[pl.ds(off, _CH1)])
        return 0

    lax.fori_loop(0, _EPW // _CH1, chunk, 0)


def _sc_scale(dst, typ, inv):
    mesh = plsc.VectorSubcoreMesh(core_axis_name="c", subcore_axis_name="s")
    return pl.kernel(
        _sc_scale_body,
        out_type=jax.ShapeDtypeStruct((_E,), jnp.float32),
        mesh=mesh,
        compiler_params=pltpu.CompilerParams(needs_layout_passes=False),
        scratch_types=[
            pltpu.VMEM((_KP,), jnp.float32),
            pltpu.VMEM((_CH1,), jnp.int32),
            pltpu.VMEM((_CH1,), jnp.int32),
            pltpu.VMEM((_CH1,), jnp.float32),
        ],
    )(dst, typ, inv)


def _sc_edge_body(src_hbm, dst_hbm, typ_hbm, scl_hbm, y_hbm, znh_hbm,
                  out_hbm, acc, rows0, rows1, gkey_v, dst0, dst1, scl_v,
                  st_s, st_t, sem0, sem1):
    cid = lax.axis_index("c")
    sid = lax.axis_index("s")
    wid = sid * _NC + cid
    base_e = wid * _EPW

    for k in range(_RPT // 32):
        rows = pl.ds(sid * _RPT + k * 32, 32)
        pltpu.sync_copy(znh_hbm.at[rows], acc.at[rows])

    pltpu.sync_copy(scl_hbm.at[pl.ds(base_e, _EPW)], scl_v)

    def sup(c2, _):
        off = base_e + c2 * _CH1
        pltpu.sync_copy(src_hbm.at[pl.ds(off, _CH1)], st_s)
        pltpu.sync_copy(typ_hbm.at[pl.ds(off, _CH1)], st_t)

        def fill(kk, _):
            for j2 in range(5):
                sl = pl.ds(kk * _CH + j2 * _L, _L)
                gkey_v[pl.ds(c2 * _CH1 + kk * _CH + j2 * _L, _L)] = (
                    st_t[sl] * _N + st_s[sl])
            return 0

        return lax.fori_loop(0, _CH1 // _CH, fill, 0)

    lax.fori_loop(0, _EPW // _CH1, sup, 0)
    plsc.subcore_barrier()

    def start_chunk(k, rows_ref, dst_ref, sem):
        off = base_e + k * _CH
        pltpu.async_copy(dst_hbm.at[pl.ds(off, _CH)], dst_ref, sem)
        pltpu.async_copy(y_hbm.at[gkey_v.at[pl.ds(k * _CH, _CH)]],
                         rows_ref, sem)

    def wait_chunk(k, rows_ref, dst_ref, sem):
        off = base_e + k * _CH
        pltpu.make_async_copy(dst_hbm.at[pl.ds(off, _CH)], dst_ref,
                              sem).wait()
        pltpu.make_async_copy(y_hbm.at[gkey_v.at[pl.ds(k * _CH, _CH)]],
                              rows_ref, sem).wait()

    def scale_scatter(k, rows_ref, dst_ref):
        for j in range(_CH // _L):
            iv = scl_v[pl.ds(k * _CH + j * _L, _L)]
            for e in range(_L):
                row = j * _L + e
                sc = _splat(iv, e)
                for q in range(_H // _L):
                    rows_ref[row, pl.ds(q * _L, _L)] = (
                        rows_ref[row, pl.ds(q * _L, _L)] * sc)
        pltpu.sync_copy(rows_ref, acc.at[dst_ref], add=True)

    start_chunk(0, rows0, dst0, sem0)
    start_chunk(1, rows1, dst1, sem1)

    def body(i, _):
        k0 = 2 * i
        k1 = 2 * i + 1
        wait_chunk(k0, rows0, dst0, sem0)
        scale_scatter(k0, rows0, dst0)

        @pl.when(k0 + 2 < _NCH)
        def _():
            start_chunk(k0 + 2, rows0, dst0, sem0)

        wait_chunk(k1, rows1, dst1, sem1)
        scale_scatter(k1, rows1, dst1)

        @pl.when(k1 + 2 < _NCH)
        def _():
            start_chunk(k1 + 2, rows1, dst1, sem1)

        return 0

    lax.fori_loop(0, _NCH // 2, body, 0)
    wait_chunk(_NCH - 1, rows0, dst0, sem0)
    scale_scatter(_NCH - 1, rows0, dst0)

    plsc.subcore_barrier()
    for k in range(_RPT // 32):
        rows = pl.ds(sid * _RPT + k * 32, 32)
        pltpu.sync_copy(acc.at[rows], out_hbm.at[cid, rows])


def _sc_edges(src, dst, typ, scl, y2, zeros_nh):
    mesh = plsc.VectorSubcoreMesh(core_axis_name="c", subcore_axis_name="s")
    return pl.kernel(
        _sc_edge_body,
        out_type=jax.ShapeDtypeStruct((_NC, _NP, _H), jnp.float32),
        mesh=mesh,
        compiler_params=pltpu.CompilerParams(needs_layout_passes=False),
        scratch_types=[
            pltpu.VMEM_SHARED((_NP, _H), jnp.float32),
            pltpu.VMEM((_CH, _H), jnp.float32),
            pltpu.VMEM((_CH, _H), jnp.float32),
            pltpu.VMEM((_EPW,), jnp.int32),
            pltpu.VMEM((_CH,), jnp.int32),
            pltpu.VMEM((_CH,), jnp.int32),
            pltpu.VMEM((_EPW,), jnp.float32),
            pltpu.VMEM((_CH1,), jnp.int32),
            pltpu.VMEM((_CH1,), jnp.int32),
            pltpu.SemaphoreType.DMA,
            pltpu.SemaphoreType.DMA,
        ],
    )(src, dst, typ, scl, y2, zeros_nh)


def _tc_combine_body(o_ref, p_ref, out_ref):
    out_ref[...] = o_ref[...] + p_ref[0] + p_ref[1]


def _tc_combine(out0, parts):
    blk = 1000
    return pl.pallas_call(
        _tc_combine_body,
        grid=(_N // blk,),
        in_specs=[
            pl.BlockSpec((blk, _H), lambda i: (i, 0)),
            pl.BlockSpec((_NC, blk, _H), lambda i: (0, i, 0)),
        ],
        out_specs=pl.BlockSpec((blk, _H), lambda i: (i, 0)),
        out_shape=jax.ShapeDtypeStruct((_N, _H), jnp.float32),
    )(out0, parts)


def kernel(input_s, input_a, edge_index, edge_type, bases, comp, root, bias):
    x = jnp.concatenate([input_s, input_a], axis=1).reshape(-1, _H)
    src = edge_index[0]
    dst = edge_index[1]
    typ = edge_type

    zeros_k = jnp.zeros((_KP,), jnp.float32)
    zeros_nh = jnp.zeros((_NP, _H), jnp.float32)

    cnt_parts = _sc_cnt(dst, typ, zeros_k)
    y, out0 = _tc_mm(x, bases, comp, root, bias)
    inv = _tc_inv(cnt_parts)
    scl = _sc_scale(dst, typ, inv)
    parts = _sc_edges(src, dst, typ, scl, y.reshape(_K, _H), zeros_nh)
    out = _tc_combine(out0, parts)

    dtrp = out.reshape(input_s.shape[0], -1, _H)
    sent, act = jnp.split(dtrp, 2, axis=1)
    return (sent, act)


# speedup vs baseline: 1.1130x; 1.1015x over previous
import functools

import jax
import jax.numpy as jnp
from jax import lax
from jax.experimental import pallas as pl
from jax.experimental.pallas import tpu as pltpu
from jax.experimental.pallas import tpu_sc as plsc

_H = 128
_R = 8
_NB = 4
_N = 10000
_E = 320000
_K = _N * _R
_KP = 81920
_NC = 2
_NS = 16
_NW = _NC * _NS
_EPW = _E // _NW
_CH = 80
_NCH = _EPW // _CH
_CH1 = 2000
_NP = 10240
_RPT = _NP // _NS
_L = 16


def _splat(vec, e):
    idx = jnp.full((_L,), e, dtype=jnp.int32)
    return lax.gather(
        vec, idx[:, None],
        dimension_numbers=lax.GatherDimensionNumbers(
            offset_dims=(), collapsed_slice_dims=(0,), start_index_map=(0,)),
        slice_sizes=(1,), mode=lax.GatherScatterMode.PROMISE_IN_BOUNDS)


def _sc_cnt_body(dst_hbm, typ_hbm, zk_hbm, out_hbm, cnt_v, dst0, typ0,
                 dst1, typ1, sem0, sem1):
    cid = lax.axis_index("c")
    sid = lax.axis_index("s")
    wid = sid * _NC + cid
    base_e = wid * _EPW

    def start(c, d_ref, t_ref, sem):
        off = base_e + c * _CH1
        pltpu.async_copy(dst_hbm.at[pl.ds(off, _CH1)], d_ref, sem)
        pltpu.async_copy(typ_hbm.at[pl.ds(off, _CH1)], t_ref, sem)

    def wait(c, d_ref, t_ref, sem):
        off = base_e + c * _CH1
        pltpu.make_async_copy(dst_hbm.at[pl.ds(off, _CH1)], d_ref,
                              sem).wait()
        pltpu.make_async_copy(typ_hbm.at[pl.ds(off, _CH1)], t_ref,
                              sem).wait()

    def accum(d_ref, t_ref):
        def group(j, _):
            key = t_ref[pl.ds(j * _L, _L)] * _N + d_ref[pl.ds(j * _L, _L)]
            plsc.addupdate_scatter(cnt_v, [key],
                                   jnp.ones((_L,), jnp.float32))
            return 0

        lax.fori_loop(0, _CH1 // _L, group, 0)

    start(0, dst0, typ0, sem0)
    pltpu.sync_copy(zk_hbm, cnt_v)
    nch = _EPW // _CH1
    for c in range(nch):
        d_ref, t_ref, sem = (dst0, typ0, sem0) if c % 2 == 0 else (
            dst1, typ1, sem1)
        if c + 1 < nch:
            n_ref, nt_ref, nsem = (dst0, typ0, sem0) if (c + 1) % 2 == 0 \
                else (dst1, typ1, sem1)
            start(c + 1, n_ref, nt_ref, nsem)
        wait(c, d_ref, t_ref, sem)
        accum(d_ref, t_ref)
    pltpu.sync_copy(cnt_v, out_hbm.at[wid])


def _sc_cnt(dst, typ, zeros_k):
    mesh = plsc.VectorSubcoreMesh(core_axis_name="c", subcore_axis_name="s")
    return pl.kernel(
        _sc_cnt_body,
        out_type=jax.ShapeDtypeStruct((_NW, _KP), jnp.float32),
        mesh=mesh,
        compiler_params=pltpu.CompilerParams(needs_layout_passes=False),
        scratch_types=[
            pltpu.VMEM((_KP,), jnp.float32),
            pltpu.VMEM((_CH1,), jnp.int32),
            pltpu.VMEM((_CH1,), jnp.int32),
            pltpu.VMEM((_CH1,), jnp.int32),
            pltpu.VMEM((_CH1,), jnp.int32),
            pltpu.SemaphoreType.DMA,
            pltpu.SemaphoreType.DMA,
        ],
    )(dst, typ, zeros_k)


def _tc_mm_body(x_ref, bases_ref, comp_ref, root_ref, bias_ref, y_ref, o_ref):
    x = x_ref[...]
    zs = [jnp.dot(x, bases_ref[b], preferred_element_type=jnp.float32)
          for b in range(_NB)]
    for r in range(_R):
        acc = zs[0] * comp_ref[r, 0]
        for b in range(1, _NB):
            acc = acc + zs[b] * comp_ref[r, b]
        y_ref[r, :, :] = acc
    o_ref[...] = (jnp.dot(x, root_ref[...], preferred_element_type=jnp.float32)
                  + bias_ref[...])


def _tc_mm(x, bases, comp, root, bias):
    blk = 1000
    return pl.pallas_call(
        _tc_mm_body,
        grid=(_N // blk,),
        in_specs=[
            pl.BlockSpec((blk, _H), lambda i: (i, 0)),
            pl.BlockSpec((_NB, _H, _H), lambda i: (0, 0, 0)),
            pl.BlockSpec(memory_space=pltpu.SMEM),
            pl.BlockSpec((_H, _H), lambda i: (0, 0)),
            pl.BlockSpec((_H,), lambda i: (0,)),
        ],
        out_specs=[
            pl.BlockSpec((_R, blk, _H), lambda i: (0, i, 0)),
            pl.BlockSpec((blk, _H), lambda i: (i, 0)),
        ],
        out_shape=[
            jax.ShapeDtypeStruct((_R, _N, _H), jnp.float32),
            jax.ShapeDtypeStruct((_N, _H), jnp.float32),
        ],
    )(x, bases, comp, root, bias)


def _tc_inv_body(p_ref, inv_ref):
    s = jnp.sum(p_ref[...], axis=0)
    inv_ref[...] = 1.0 / jnp.maximum(s, 1.0)


def _tc_inv(parts):
    blk = 16384
    return pl.pallas_call(
        _tc_inv_body,
        grid=(_KP // blk,),
        in_specs=[pl.BlockSpec((_NW, blk), lambda i: (0, i))],
        out_specs=pl.BlockSpec((blk,), lambda i: (i,)),
        out_shape=jax.ShapeDtypeStruct((_KP,), jnp.float32),
    )(parts)


def _sc_scale_body(dst_hbm, typ_hbm, inv_hbm, out_hbm, inv_v, dst0, typ0,
                   dst1, typ1, scl_v, sem0, sem1):
    cid = lax.axis_index("c")
    sid = lax.axis_index("s")
    wid = sid * _NC + cid
    base_e = wid * _EPW

    def start(c, d_ref, t_ref, sem):
        off = base_e + c * _CH1
        pltpu.async_copy(dst_hbm.at[pl.ds(off, _CH1)], d_ref, sem)
        pltpu.async_copy(typ_hbm.at[pl.ds(off, _CH1)], t_ref, sem)

    def wait(c, d_ref, t_ref, sem):
        off = base_e + c * _CH1
        pltpu.make_async_copy(dst_hbm.at[pl.ds(off, _CH1)], d_ref,
                              sem).wait()
        pltpu.make_async_copy(typ_hbm.at[pl.ds(off, _CH1)], t_ref,
                              sem).wait()

    start(0, dst0, typ0, sem0)
    pltpu.sync_copy(inv_hbm, inv_v)
    nch = _EPW // _CH1
    for c in range(nch):
        d_ref, t_ref, sem = (dst0, typ0, sem0) if c % 2 == 0 else (
            dst1, typ1, sem1)
        if c + 1 < nch:
            n_ref, nt_ref, nsem = (dst0, typ0, sem0) if (c + 1) % 2 == 0 \
                else (dst1, typ1, sem1)
            start(c + 1, n_ref, nt_ref, nsem)
        wait(c, d_ref, t_ref, sem)

        def group(j, _):
            sl = pl.ds(j * _L, _L)
            key = t_ref[sl] * _N + d_ref[sl]
            scl_v[sl] = plsc.load_gather(inv_v, [key])
            return 0

        lax.fori_loop(0, _CH1 // _L, group, 0)
        pltpu.sync_copy(scl_v, out_hbm.at[pl.ds(base_e + c * _CH1, _CH1)])


def _sc_scale(dst, typ, inv):
    mesh = plsc.VectorSubcoreMesh(core_axis_name="c", subcore_axis_name="s")
    return pl.kernel(
        _sc_scale_body,
        out_type=jax.ShapeDtypeStruct((_E,), jnp.float32),
        mesh=mesh,
        compiler_params=pltpu.CompilerParams(needs_layout_passes=False),
        scratch_types=[
            pltpu.VMEM((_KP,), jnp.float32),
            pltpu.VMEM((_CH1,), jnp.int32),
            pltpu.VMEM((_CH1,), jnp.int32),
            pltpu.VMEM((_CH1,), jnp.int32),
            pltpu.VMEM((_CH1,), jnp.int32),
            pltpu.VMEM((_CH1,), jnp.float32),
            pltpu.SemaphoreType.DMA,
            pltpu.SemaphoreType.DMA,
        ],
    )(dst, typ, inv)


def _sc_edge_body(src_hbm, dst_hbm, typ_hbm, scl_hbm, y_hbm, znh_hbm,
                  out_hbm, acc, rows0, rows1, gkey_v, dst0, dst1, scl0,
                  scl1, st_s, st_t, sem0, sem1):
    cid = lax.axis_index("c")
    sid = lax.axis_index("s")
    wid = sid * _NC + cid
    base_e = wid * _EPW

    for k in range(_RPT // 64):
        rows = pl.ds(sid * _RPT + k * 64, 64)
        pltpu.sync_copy(znh_hbm.at[rows], acc.at[rows])

    def sup(c2, _):
        off = base_e + c2 * _CH1
        pltpu.sync_copy(src_hbm.at[pl.ds(off, _CH1)], st_s)
        pltpu.sync_copy(typ_hbm.at[pl.ds(off, _CH1)], st_t)

        def fill(kk, _):
            for j2 in range(5):
                sl = pl.ds(kk * _CH + j2 * _L, _L)
                gkey_v[pl.ds(c2 * _CH1 + kk * _CH + j2 * _L, _L)] = (
                    st_t[sl] * _N + st_s[sl])
            return 0

        return lax.fori_loop(0, _CH1 // _CH, fill, 0)

    lax.fori_loop(0, _EPW // _CH1, sup, 0)
    plsc.subcore_barrier()

    def start_chunk(k, rows_ref, dst_ref, scl_ref, sem):
        off = base_e + k * _CH
        pltpu.async_copy(dst_hbm.at[pl.ds(off, _CH)], dst_ref, sem)
        pltpu.async_copy(scl_hbm.at[pl.ds(off, _CH)], scl_ref, sem)
        pltpu.async_copy(y_hbm.at[gkey_v.at[pl.ds(k * _CH, _CH)]],
                         rows_ref, sem)

    def wait_chunk(k, rows_ref, dst_ref, scl_ref, sem):
        off = base_e + k * _CH
        pltpu.make_async_copy(dst_hbm.at[pl.ds(off, _CH)], dst_ref,
                              sem).wait()
        pltpu.make_async_copy(scl_hbm.at[pl.ds(off, _CH)], scl_ref,
                              sem).wait()
        pltpu.make_async_copy(y_hbm.at[gkey_v.at[pl.ds(k * _CH, _CH)]],
                              rows_ref, sem).wait()

    def scale_scatter(rows_ref, dst_ref, scl_ref):
        for j in range(_CH // _L):
            iv = scl_ref[pl.ds(j * _L, _L)]
            for e in range(_L):
                row = j * _L + e
                sc = _splat(iv, e)
                for q in range(_H // _L):
                    rows_ref[row, pl.ds(q * _L, _L)] = (
                        rows_ref[row, pl.ds(q * _L, _L)] * sc)
        pltpu.sync_copy(rows_ref, acc.at[dst_ref], add=True)

    start_chunk(0, rows0, dst0, scl0, sem0)
    start_chunk(1, rows1, dst1, scl1, sem1)

    def body(i, _):
        k0 = 2 * i
        k1 = 2 * i + 1
        wait_chunk(k0, rows0, dst0, scl0, sem0)
        scale_scatter(rows0, dst0, scl0)

        @pl.when(k0 + 2 < _NCH)
        def _():
            start_chunk(k0 + 2, rows0, dst0, scl0, sem0)

        wait_chunk(k1, rows1, dst1, scl1, sem1)
        scale_scatter(rows1, dst1, scl1)

        @pl.when(k1 + 2 < _NCH)
        def _():
            start_chunk(k1 + 2, rows1, dst1, scl1, sem1)

        return 0

    lax.fori_loop(0, _NCH // 2, body, 0)
    wait_chunk(_NCH - 1, rows0, dst0, scl0, sem0)
    scale_scatter(rows0, dst0, scl0)

    plsc.subcore_barrier()
    for k in range(_RPT // 64):
        rows = pl.ds(sid * _RPT + k * 64, 64)
        pltpu.sync_copy(acc.at[rows], out_hbm.at[cid, rows])


def _sc_edges(src, dst, typ, scl, y2, zeros_nh):
    mesh = plsc.VectorSubcoreMesh(core_axis_name="c", subcore_axis_name="s")
    return pl.kernel(
        _sc_edge_body,
        out_type=jax.ShapeDtypeStruct((_NC, _NP, _H), jnp.float32),
        mesh=mesh,
        compiler_params=pltpu.CompilerParams(needs_layout_passes=False),
        scratch_types=[
            pltpu.VMEM_SHARED((_NP, _H), jnp.float32),
            pltpu.VMEM((_CH, _H), jnp.float32),
            pltpu.VMEM((_CH, _H), jnp.float32),
            pltpu.VMEM((_EPW,), jnp.int32),
            pltpu.VMEM((_CH,), jnp.int32),
            pltpu.VMEM((_CH,), jnp.int32),
            pltpu.VMEM((_CH,), jnp.float32),
            pltpu.VMEM((_CH,), jnp.float32),
            pltpu.VMEM((_CH1,), jnp.int32),
            pltpu.VMEM((_CH1,), jnp.int32),
            pltpu.SemaphoreType.DMA,
            pltpu.SemaphoreType.DMA,
        ],
    )(src, dst, typ, scl, y2, zeros_nh)


def _tc_combine_body(o_ref, p_ref, out_ref):
    out_ref[...] = o_ref[...] + p_ref[0] + p_ref[1]


def _tc_combine(out0, parts):
    blk = 1000
    return pl.pallas_call(
        _tc_combine_body,
        grid=(_N // blk,),
        in_specs=[
            pl.BlockSpec((blk, _H), lambda i: (i, 0)),
            pl.BlockSpec((_NC, blk, _H), lambda i: (0, i, 0)),
        ],
        out_specs=pl.BlockSpec((blk, _H), lambda i: (i, 0)),
        out_shape=jax.ShapeDtypeStruct((_N, _H), jnp.float32),
    )(out0, parts)


def kernel(input_s, input_a, edge_index, edge_type, bases, comp, root, bias):
    x = jnp.concatenate([input_s, input_a], axis=1).reshape(-1, _H)
    src = edge_index[0]
    dst = edge_index[1]
    typ = edge_type

    zeros_k = jnp.zeros((_KP,), jnp.float32)
    zeros_nh = jnp.zeros((_NP, _H), jnp.float32)

    cnt_parts = _sc_cnt(dst, typ, zeros_k)
    y, out0 = _tc_mm(x, bases, comp, root, bias)
    inv = _tc_inv(cnt_parts)
    scl = _sc_scale(dst, typ, inv)
    parts = _sc_edges(src, dst, typ, scl, y.reshape(_K, _H), zeros_nh)
    out = _tc_combine(out0, parts)

    dtrp = out.reshape(input_s.shape[0], -1, _H)
    sent, act = jnp.split(dtrp, 2, axis=1)
    return (sent, act)


# speedup vs baseline: 1.1357x; 1.0204x over previous
import functools

import jax
import jax.numpy as jnp
from jax import lax
from jax.experimental import pallas as pl
from jax.experimental.pallas import tpu as pltpu
from jax.experimental.pallas import tpu_sc as plsc

_H = 128
_R = 8
_NB = 4
_N = 10000
_E = 320000
_K = _N * _R
_KP = 81920
_NC = 2
_NS = 16
_NW = _NC * _NS
_EPW = _E // _NW
_CH = 80
_NCH = _EPW // _CH
_CH1 = 2000
_NP = 10240
_RPT = _NP // _NS
_L = 16


def _splat(vec, e):
    idx = jnp.full((_L,), e, dtype=jnp.int32)
    return lax.gather(
        vec, idx[:, None],
        dimension_numbers=lax.GatherDimensionNumbers(
            offset_dims=(), collapsed_slice_dims=(0,), start_index_map=(0,)),
        slice_sizes=(1,), mode=lax.GatherScatterMode.PROMISE_IN_BOUNDS)


def _sc_cnt_body(dst_hbm, typ_hbm, zk_hbm, out_hbm, cnt_v, dst0, typ0,
                 dst1, typ1, sem0, sem1):
    cid = lax.axis_index("c")
    sid = lax.axis_index("s")
    wid = sid * _NC + cid
    base_e = wid * _EPW

    def start(c, d_ref, t_ref, sem):
        off = base_e + c * _CH1
        pltpu.async_copy(dst_hbm.at[pl.ds(off, _CH1)], d_ref, sem)
        pltpu.async_copy(typ_hbm.at[pl.ds(off, _CH1)], t_ref, sem)

    def wait(c, d_ref, t_ref, sem):
        off = base_e + c * _CH1
        pltpu.make_async_copy(dst_hbm.at[pl.ds(off, _CH1)], d_ref,
                              sem).wait()
        pltpu.make_async_copy(typ_hbm.at[pl.ds(off, _CH1)], t_ref,
                              sem).wait()

    def accum(d_ref, t_ref):
        def group(j, _):
            key = t_ref[pl.ds(j * _L, _L)] * _N + d_ref[pl.ds(j * _L, _L)]
            plsc.addupdate_scatter(cnt_v, [key],
                                   jnp.ones((_L,), jnp.float32))
            return 0

        lax.fori_loop(0, _CH1 // _L, group, 0)

    start(0, dst0, typ0, sem0)
    pltpu.sync_copy(zk_hbm, cnt_v)
    nch = _EPW // _CH1
    for c in range(nch):
        d_ref, t_ref, sem = (dst0, typ0, sem0) if c % 2 == 0 else (
            dst1, typ1, sem1)
        if c + 1 < nch:
            n_ref, nt_ref, nsem = (dst0, typ0, sem0) if (c + 1) % 2 == 0 \
                else (dst1, typ1, sem1)
            start(c + 1, n_ref, nt_ref, nsem)
        wait(c, d_ref, t_ref, sem)
        accum(d_ref, t_ref)
    pltpu.sync_copy(cnt_v, out_hbm.at[wid])


def _sc_cnt(dst, typ, zeros_k):
    mesh = plsc.VectorSubcoreMesh(core_axis_name="c", subcore_axis_name="s")
    return pl.kernel(
        _sc_cnt_body,
        out_type=jax.ShapeDtypeStruct((_NW, _KP), jnp.float32),
        mesh=mesh,
        compiler_params=pltpu.CompilerParams(needs_layout_passes=False),
        scratch_types=[
            pltpu.VMEM((_KP,), jnp.float32),
            pltpu.VMEM((_CH1,), jnp.int32),
            pltpu.VMEM((_CH1,), jnp.int32),
            pltpu.VMEM((_CH1,), jnp.int32),
            pltpu.VMEM((_CH1,), jnp.int32),
            pltpu.SemaphoreType.DMA,
            pltpu.SemaphoreType.DMA,
        ],
    )(dst, typ, zeros_k)


def _tc_mm_body(x_ref, bases_ref, comp_ref, root_ref, bias_ref, y_ref, o_ref):
    x = x_ref[...]
    zs = [jnp.dot(x, bases_ref[b], preferred_element_type=jnp.float32)
          for b in range(_NB)]
    for r in range(_R):
        acc = zs[0] * comp_ref[r, 0]
        for b in range(1, _NB):
            acc = acc + zs[b] * comp_ref[r, b]
        y_ref[r, :, :] = acc
    o_ref[...] = (jnp.dot(x, root_ref[...], preferred_element_type=jnp.float32)
                  + bias_ref[...])


def _tc_mm(x, bases, comp, root, bias):
    blk = 1000
    return pl.pallas_call(
        _tc_mm_body,
        grid=(_N // blk,),
        in_specs=[
            pl.BlockSpec((blk, _H), lambda i: (i, 0)),
            pl.BlockSpec((_NB, _H, _H), lambda i: (0, 0, 0)),
            pl.BlockSpec(memory_space=pltpu.SMEM),
            pl.BlockSpec((_H, _H), lambda i: (0, 0)),
            pl.BlockSpec((_H,), lambda i: (0,)),
        ],
        out_specs=[
            pl.BlockSpec((_R, blk, _H), lambda i: (0, i, 0)),
            pl.BlockSpec((blk, _H), lambda i: (i, 0)),
        ],
        out_shape=[
            jax.ShapeDtypeStruct((_R, _N, _H), jnp.float32),
            jax.ShapeDtypeStruct((_N, _H), jnp.float32),
        ],
    )(x, bases, comp, root, bias)


def _tc_inv_body(p_ref, inv_ref):
    s = jnp.sum(p_ref[...], axis=0)
    inv_ref[...] = 1.0 / jnp.maximum(s, 1.0)


def _tc_inv(parts):
    blk = 16384
    return pl.pallas_call(
        _tc_inv_body,
        grid=(_KP // blk,),
        in_specs=[pl.BlockSpec((_NW, blk), lambda i: (0, i))],
        out_specs=pl.BlockSpec((blk,), lambda i: (i,)),
        out_shape=jax.ShapeDtypeStruct((_KP,), jnp.float32),
    )(parts)


def _sc_scale_body(src_hbm, dst_hbm, typ_hbm, inv_hbm, out_hbm, gk_hbm,
                   inv_v, dst0, typ0, src0, dst1, typ1, src1, scl_v, gk_v,
                   sem0, sem1):
    cid = lax.axis_index("c")
    sid = lax.axis_index("s")
    wid = sid * _NC + cid
    base_e = wid * _EPW

    def start(c, d_ref, t_ref, s_ref, sem):
        off = base_e + c * _CH1
        pltpu.async_copy(dst_hbm.at[pl.ds(off, _CH1)], d_ref, sem)
        pltpu.async_copy(typ_hbm.at[pl.ds(off, _CH1)], t_ref, sem)
        pltpu.async_copy(src_hbm.at[pl.ds(off, _CH1)], s_ref, sem)

    def wait(c, d_ref, t_ref, s_ref, sem):
        off = base_e + c * _CH1
        pltpu.make_async_copy(dst_hbm.at[pl.ds(off, _CH1)], d_ref,
                              sem).wait()
        pltpu.make_async_copy(typ_hbm.at[pl.ds(off, _CH1)], t_ref,
                              sem).wait()
        pltpu.make_async_copy(src_hbm.at[pl.ds(off, _CH1)], s_ref,
                              sem).wait()

    start(0, dst0, typ0, src0, sem0)
    pltpu.sync_copy(inv_hbm, inv_v)
    nch = _EPW // _CH1
    for c in range(nch):
        d_ref, t_ref, s_ref, sem = (dst0, typ0, src0, sem0) if c % 2 == 0 \
            else (dst1, typ1, src1, sem1)
        if c + 1 < nch:
            n_d, n_t, n_s, nsem = (dst0, typ0, src0, sem0) \
                if (c + 1) % 2 == 0 else (dst1, typ1, src1, sem1)
            start(c + 1, n_d, n_t, n_s, nsem)
        wait(c, d_ref, t_ref, s_ref, sem)

        def group(j, _):
            sl = pl.ds(j * _L, _L)
            t = t_ref[sl]
            scl_v[sl] = plsc.load_gather(inv_v, [t * _N + d_ref[sl]])
            gk_v[sl] = t * _N + s_ref[sl]
            return 0

        lax.fori_loop(0, _CH1 // _L, group, 0)
        pltpu.sync_copy(scl_v, out_hbm.at[pl.ds(base_e + c * _CH1, _CH1)])
        pltpu.sync_copy(gk_v, gk_hbm.at[pl.ds(base_e + c * _CH1, _CH1)])


def _sc_scale(src, dst, typ, inv):
    mesh = plsc.VectorSubcoreMesh(core_axis_name="c", subcore_axis_name="s")
    return pl.kernel(
        _sc_scale_body,
        out_type=[jax.ShapeDtypeStruct((_E,), jnp.float32),
                  jax.ShapeDtypeStruct((_E,), jnp.int32)],
        mesh=mesh,
        compiler_params=pltpu.CompilerParams(needs_layout_passes=False),
        scratch_types=[
            pltpu.VMEM((_KP,), jnp.float32),
            pltpu.VMEM((_CH1,), jnp.int32),
            pltpu.VMEM((_CH1,), jnp.int32),
            pltpu.VMEM((_CH1,), jnp.int32),
            pltpu.VMEM((_CH1,), jnp.int32),
            pltpu.VMEM((_CH1,), jnp.int32),
            pltpu.VMEM((_CH1,), jnp.int32),
            pltpu.VMEM((_CH1,), jnp.float32),
            pltpu.VMEM((_CH1,), jnp.int32),
            pltpu.SemaphoreType.DMA,
            pltpu.SemaphoreType.DMA,
        ],
    )(src, dst, typ, inv)


def _sc_edge_body(gk_hbm, dst_hbm, scl_hbm, y_hbm, znh_hbm,
                  out_hbm, acc, rows0, rows1, gkey_v, dst0, dst1, scl0,
                  scl1, sem0, sem1):
    cid = lax.axis_index("c")
    sid = lax.axis_index("s")
    wid = sid * _NC + cid
    base_e = wid * _EPW

    for k in range(_RPT // 64):
        rows = pl.ds(sid * _RPT + k * 64, 64)
        pltpu.sync_copy(znh_hbm.at[rows], acc.at[rows])

    pltpu.sync_copy(gk_hbm.at[pl.ds(base_e, _EPW)], gkey_v)
    plsc.subcore_barrier()

    def start_chunk(k, rows_ref, dst_ref, scl_ref, sem):
        off = base_e + k * _CH
        pltpu.async_copy(dst_hbm.at[pl.ds(off, _CH)], dst_ref, sem)
        pltpu.async_copy(scl_hbm.at[pl.ds(off, _CH)], scl_ref, sem)
        pltpu.async_copy(y_hbm.at[gkey_v.at[pl.ds(k * _CH, _CH)]],
                         rows_ref, sem)

    def wait_chunk(k, rows_ref, dst_ref, scl_ref, sem):
        off = base_e + k * _CH
        pltpu.make_async_copy(dst_hbm.at[pl.ds(off, _CH)], dst_ref,
                              sem).wait()
        pltpu.make_async_copy(scl_hbm.at[pl.ds(off, _CH)], scl_ref,
                              sem).wait()
        pltpu.make_async_copy(y_hbm.at[gkey_v.at[pl.ds(k * _CH, _CH)]],
                              rows_ref, sem).wait()

    def scale_scatter(rows_ref, dst_ref, scl_ref):
        for j in range(_CH // _L):
            iv = scl_ref[pl.ds(j * _L, _L)]
            for e in range(_L):
                row = j * _L + e
                sc = _splat(iv, e)
                for q in range(_H // _L):
                    rows_ref[row, pl.ds(q * _L, _L)] = (
                        rows_ref[row, pl.ds(q * _L, _L)] * sc)
        pltpu.sync_copy(rows_ref, acc.at[dst_ref], add=True)

    start_chunk(0, rows0, dst0, scl0, sem0)
    start_chunk(1, rows1, dst1, scl1, sem1)

    def body(i, _):
        k0 = 2 * i
        k1 = 2 * i + 1
        wait_chunk(k0, rows0, dst0, scl0, sem0)
        scale_scatter(rows0, dst0, scl0)

        @pl.when(k0 + 2 < _NCH)
        def _():
            start_chunk(k0 + 2, rows0, dst0, scl0, sem0)

        wait_chunk(k1, rows1, dst1, scl1, sem1)
        scale_scatter(rows1, dst1, scl1)

        @pl.when(k1 + 2 < _NCH)
        def _():
            start_chunk(k1 + 2, rows1, dst1, scl1, sem1)

        return 0

    lax.fori_loop(0, _NCH // 2, body, 0)
    wait_chunk(_NCH - 1, rows0, dst0, scl0, sem0)
    scale_scatter(rows0, dst0, scl0)

    plsc.subcore_barrier()
    for k in range(_RPT // 64):
        rows = pl.ds(sid * _RPT + k * 64, 64)
        pltpu.sync_copy(acc.at[rows], out_hbm.at[cid, rows])


def _sc_edges(gk, dst, scl, y2, zeros_nh):
    mesh = plsc.VectorSubcoreMesh(core_axis_name="c", subcore_axis_name="s")
    return pl.kernel(
        _sc_edge_body,
        out_type=jax.ShapeDtypeStruct((_NC, _NP, _H), jnp.float32),
        mesh=mesh,
        compiler_params=pltpu.CompilerParams(needs_layout_passes=False),
        scratch_types=[
            pltpu.VMEM_SHARED((_NP, _H), jnp.float32),
            pltpu.VMEM((_CH, _H), jnp.float32),
            pltpu.VMEM((_CH, _H), jnp.float32),
            pltpu.VMEM((_EPW,), jnp.int32),
            pltpu.VMEM((_CH,), jnp.int32),
            pltpu.VMEM((_CH,), jnp.int32),
            pltpu.VMEM((_CH,), jnp.float32),
            pltpu.VMEM((_CH,), jnp.float32),
            pltpu.SemaphoreType.DMA,
            pltpu.SemaphoreType.DMA,
        ],
    )(gk, dst, scl, y2, zeros_nh)


def _tc_combine_body(o_ref, p_ref, out_ref):
    out_ref[...] = o_ref[...] + p_ref[0] + p_ref[1]


def _tc_combine(out0, parts):
    blk = 1000
    return pl.pallas_call(
        _tc_combine_body,
        grid=(_N // blk,),
        in_specs=[
            pl.BlockSpec((blk, _H), lambda i: (i, 0)),
            pl.BlockSpec((_NC, blk, _H), lambda i: (0, i, 0)),
        ],
        out_specs=pl.BlockSpec((blk, _H), lambda i: (i, 0)),
        out_shape=jax.ShapeDtypeStruct((_N, _H), jnp.float32),
    )(out0, parts)


def kernel(input_s, input_a, edge_index, edge_type, bases, comp, root, bias):
    x = jnp.concatenate([input_s, input_a], axis=1).reshape(-1, _H)
    src = edge_index[0]
    dst = edge_index[1]
    typ = edge_type

    zeros_k = jnp.zeros((_KP,), jnp.float32)
    zeros_nh = jnp.zeros((_NP, _H), jnp.float32)

    cnt_parts = _sc_cnt(dst, typ, zeros_k)
    y, out0 = _tc_mm(x, bases, comp, root, bias)
    inv = _tc_inv(cnt_parts)
    scl, gk = _sc_scale(src, dst, typ, inv)
    parts = _sc_edges(gk, dst, scl, y.reshape(_K, _H), zeros_nh)
    out = _tc_combine(out0, parts)

    dtrp = out.reshape(input_s.shape[0], -1, _H)
    sent, act = jnp.split(dtrp, 2, axis=1)
    return (sent, act)
